# baseline (speedup 1.0000x reference)
import functools

import jax
import jax.numpy as jnp
from jax import lax
from jax.experimental import pallas as pl
from jax.experimental.pallas import tpu as pltpu
from jax.experimental.pallas import tpu_sc as plsc

N_USER = 6000
N_ITEM = 4000
N = N_USER + N_ITEM
D = 128
E = 320000
NC = 2
NS = 16
NW = NC * NS
CH = 96
NCHUNK = 105
EPW = NCHUNK * CH
EPAD = NW * EPW - E
NPAD = 10240
RPS = NPAD // NS

_mesh = plsc.VectorSubcoreMesh(core_axis_name="c", subcore_axis_name="s")


@functools.partial(
    pl.kernel,
    mesh=_mesh,
    compiler_params=pltpu.CompilerParams(needs_layout_passes=False),
    out_type=(
        jax.ShapeDtypeStruct((NC, NPAD, D), jnp.float32),
        jax.ShapeDtypeStruct((NW, NPAD), jnp.float32),
    ),
    scratch_types=[
        pltpu.VMEM((4, 2, CH), jnp.int32),
        pltpu.VMEM((2, CH, D), jnp.float32),
        pltpu.VMEM((NPAD,), jnp.float32),
        pltpu.VMEM_SHARED((NPAD, D), jnp.float32),
        pltpu.SemaphoreType.DMA,
        pltpu.SemaphoreType.DMA,
        pltpu.SemaphoreType.DMA,
        pltpu.SemaphoreType.DMA,
        pltpu.SemaphoreType.DMA,
        pltpu.SemaphoreType.DMA,
    ],
)
def _sc_segment_sum(adj_hbm, table_hbm, zeros_hbm, zerosd_hbm,
                    out_hbm, outdeg_hbm,
                    idx_v, rows_v, deg_v, acc_sh,
                    gsem0, gsem1, isem0, isem1, isem2, isem3):
    c = lax.axis_index("c")
    s = lax.axis_index("s")
    wid = c * NS + s
    pltpu.sync_copy(zeros_hbm, acc_sh.at[pl.ds(s * RPS, RPS)])
    pltpu.sync_copy(zerosd_hbm, deg_v)
    plsc.subcore_barrier()

    ones16 = jnp.full((16,), 1.0, jnp.float32)
    gsems = (gsem0, gsem1)
    isems = (isem0, isem1, isem2, isem3)

    def fire_idx(j, m):
        pltpu.async_copy(adj_hbm.at[0, wid, j], idx_v.at[m, 0], isems[m])
        pltpu.async_copy(adj_hbm.at[1, wid, j], idx_v.at[m, 1], isems[m])

    def fire_gather(j, m, r):
        pltpu.make_async_copy(adj_hbm.at[0, wid, j], idx_v.at[m, 0], isems[m]).wait()
        pltpu.make_async_copy(adj_hbm.at[1, wid, j], idx_v.at[m, 1], isems[m]).wait()
        pltpu.async_copy(table_hbm.at[idx_v.at[m, 0]], rows_v.at[r], gsems[r])
        for k in range(CH // 16):
            idx16 = idx_v[m, 1, pl.ds(k * 16, 16)]
            plsc.addupdate_scatter(deg_v, [idx16], ones16)

    def finish(j, m, r):
        pltpu.make_async_copy(
            table_hbm.at[idx_v.at[m, 0]], rows_v.at[r], gsems[r]).wait()
        pltpu.sync_copy(rows_v.at[r], acc_sh.at[idx_v.at[m, 1]], add=True)

        @pl.when(j + 4 < NCHUNK)
        def _():
            fire_idx(j + 4, m)

    for m in range(4):
        fire_idx(m, m)
    fire_gather(0, 0, 0)

    def body(k, carry):
        base = 4 * k
        fire_gather(base + 1, 1, 1)
        finish(base, 0, 0)
        fire_gather(base + 2, 2, 0)
        finish(base + 1, 1, 1)
        fire_gather(base + 3, 3, 1)
        finish(base + 2, 2, 0)
        fire_gather(base + 4, 0, 0)
        finish(base + 3, 3, 1)
        return carry

    lax.fori_loop(0, NCHUNK // 4, body, 0)
    finish(NCHUNK - 1, 0, 0)
    pltpu.sync_copy(deg_v, outdeg_hbm.at[wid])
    plsc.subcore_barrier()
    pltpu.sync_copy(acc_sh.at[pl.ds(s * RPS, RPS)],
                    out_hbm.at[c, pl.ds(s * RPS, RPS)])


def _tc_epilogue(x_ref, parts_ref, degp_ref, w_ref, t_ref, out_ref):
    x = x_ref[...]
    agg = parts_ref[0] + parts_ref[1]
    inv_nrm = lax.rsqrt(jnp.max(jnp.sum(x * x, axis=1)))
    deg = jnp.sum(jnp.transpose(degp_ref[...]), axis=1, keepdims=True)
    scale = lax.rsqrt(deg + 1.0) * inv_nrm
    aggn = agg * scale
    z = jnp.tanh(jnp.dot(aggn, w_ref[...], preferred_element_type=jnp.float32))
    out_ref[...] = x * inv_nrm + t_ref[0, 0] * z


def kernel(adj_his, t_diff, xu_t_plus, xi_t_plus, xu_embed, xi_embed, W):
    x_t_plus = jnp.concatenate(
        [xu_t_plus, xi_t_plus, jnp.zeros((NPAD - N, D), jnp.float32)], axis=0)
    table = jnp.concatenate([xu_embed, xi_embed], axis=0)
    adj32 = adj_his.astype(jnp.int32)
    pad_dst = N + (jnp.arange(EPAD, dtype=jnp.int32) % (NPAD - N))
    pad = jnp.stack([jnp.zeros((EPAD,), jnp.int32), pad_dst], axis=0)
    adj = jnp.concatenate([adj32, pad], axis=1).reshape(2, NW, NCHUNK, CH)
    zeros = jnp.zeros((RPS, D), jnp.float32)
    zerosd = jnp.zeros((NPAD,), jnp.float32)

    parts, deg_parts = _sc_segment_sum(adj, table, zeros, zerosd)

    out = pl.pallas_call(
        _tc_epilogue,
        out_shape=jax.ShapeDtypeStruct((NPAD, D), jnp.float32),
        in_specs=[
            pl.BlockSpec(memory_space=pltpu.VMEM),
            pl.BlockSpec(memory_space=pltpu.VMEM),
            pl.BlockSpec(memory_space=pltpu.VMEM),
            pl.BlockSpec(memory_space=pltpu.VMEM),
            pl.BlockSpec(memory_space=pltpu.SMEM),
        ],
        out_specs=pl.BlockSpec(memory_space=pltpu.VMEM),
    )(x_t_plus, parts, deg_parts, W, t_diff.reshape(1, 1))

    return (out[:N_USER], out[N_USER:N])


# speedup vs baseline: 1.5880x; 1.5880x over previous
import functools

import jax
import jax.numpy as jnp
from jax import lax
from jax.experimental import pallas as pl
from jax.experimental.pallas import tpu as pltpu
from jax.experimental.pallas import tpu_sc as plsc

N_USER = 6000
N_ITEM = 4000
N = N_USER + N_ITEM
D = 128
E = 320000
NC = 2
NS = 16
NW = NC * NS
EPW = E // NW
CH = 80
NCHUNK = EPW // CH
NPAD = 10240
RPS = NPAD // NS

_mesh = plsc.VectorSubcoreMesh(core_axis_name="c", subcore_axis_name="s")


@functools.partial(
    pl.kernel,
    mesh=_mesh,
    compiler_params=pltpu.CompilerParams(needs_layout_passes=False),
    out_type=(
        jax.ShapeDtypeStruct((NC, NPAD, D), jnp.float32),
        jax.ShapeDtypeStruct((NW, NPAD), jnp.float32),
    ),
    scratch_types=[
        pltpu.VMEM((4, 2, CH), jnp.int32),
        pltpu.VMEM((2, CH, D), jnp.float32),
        pltpu.VMEM((NPAD,), jnp.float32),
        pltpu.VMEM_SHARED((NPAD, D), jnp.float32),
        pltpu.SemaphoreType.DMA,
        pltpu.SemaphoreType.DMA,
        pltpu.SemaphoreType.DMA,
        pltpu.SemaphoreType.DMA,
        pltpu.SemaphoreType.DMA,
        pltpu.SemaphoreType.DMA,
    ],
)
def _sc_segment_sum(adj_hbm, table_hbm, zeros_hbm, zerosd_hbm,
                    out_hbm, outdeg_hbm,
                    idx_v, rows_v, deg_v, acc_sh,
                    gsem0, gsem1, isem0, isem1, isem2, isem3):
    c = lax.axis_index("c")
    s = lax.axis_index("s")
    wid = c * NS + s
    pltpu.sync_copy(zeros_hbm, acc_sh.at[pl.ds(s * RPS, RPS)])
    pltpu.sync_copy(zerosd_hbm, deg_v)
    plsc.subcore_barrier()

    ones16 = jnp.full((16,), 1.0, jnp.float32)
    gsems = (gsem0, gsem1)
    isems = (isem0, isem1, isem2, isem3)

    def fire_idx(j, m):
        pltpu.async_copy(adj_hbm.at[0, wid, j], idx_v.at[m, 0], isems[m])
        pltpu.async_copy(adj_hbm.at[1, wid, j], idx_v.at[m, 1], isems[m])

    def fire_gather(j, m, r):
        pltpu.make_async_copy(adj_hbm.at[0, wid, j], idx_v.at[m, 0], isems[m]).wait()
        pltpu.make_async_copy(adj_hbm.at[1, wid, j], idx_v.at[m, 1], isems[m]).wait()
        pltpu.async_copy(table_hbm.at[idx_v.at[m, 0]], rows_v.at[r], gsems[r])
        for k in range(CH // 16):
            idx16 = idx_v[m, 1, pl.ds(k * 16, 16)]
            plsc.addupdate_scatter(deg_v, [idx16], ones16)

    def finish(j, m, r):
        pltpu.make_async_copy(
            table_hbm.at[idx_v.at[m, 0]], rows_v.at[r], gsems[r]).wait()
        pltpu.sync_copy(rows_v.at[r], acc_sh.at[idx_v.at[m, 1]], add=True)

        @pl.when(j + 4 < NCHUNK)
        def _():
            fire_idx(j + 4, m)

    for m in range(4):
        fire_idx(m, m)
    fire_gather(0, 0, 0)

    def body(k, carry):
        base = 4 * k
        fire_gather(base + 1, 1, 1)
        finish(base, 0, 0)
        fire_gather(base + 2, 2, 0)
        finish(base + 1, 1, 1)
        fire_gather(base + 3, 3, 1)
        finish(base + 2, 2, 0)
        fire_gather(base + 4, 0, 0)
        finish(base + 3, 3, 1)
        return carry

    lax.fori_loop(0, NCHUNK // 4, body, 0)
    finish(NCHUNK - 1, 0, 0)
    pltpu.sync_copy(deg_v, outdeg_hbm.at[wid])
    plsc.subcore_barrier()
    pltpu.sync_copy(acc_sh.at[pl.ds(s * RPS, RPS)],
                    out_hbm.at[c, pl.ds(s * RPS, RPS)])


def _tc_epilogue(xu_ref, xi_ref, parts_ref, degp_ref, w_ref, t_ref,
                 outu_ref, outi_ref):
    xu = xu_ref[...]
    xi = xi_ref[...]
    nrm2 = jnp.maximum(jnp.max(jnp.sum(xu * xu, axis=1)),
                       jnp.max(jnp.sum(xi * xi, axis=1)))
    inv_nrm = lax.rsqrt(nrm2)
    agg = parts_ref[0] + parts_ref[1]
    deg = jnp.sum(jnp.transpose(degp_ref[...]), axis=1, keepdims=True)
    scale = lax.rsqrt(deg + 1.0) * inv_nrm
    aggn = agg * scale
    w = w_ref[...]
    t = t_ref[0, 0]
    zu = jnp.tanh(jnp.dot(aggn[:N_USER], w,
                          preferred_element_type=jnp.float32))
    outu_ref[...] = xu * inv_nrm + t * zu
    zi = jnp.tanh(jnp.dot(aggn[N_USER:N], w,
                          preferred_element_type=jnp.float32))
    outi_ref[...] = xi * inv_nrm + t * zi


def kernel(adj_his, t_diff, xu_t_plus, xi_t_plus, xu_embed, xi_embed, W):
    table = jnp.concatenate([xu_embed, xi_embed], axis=0)
    adj32 = adj_his.astype(jnp.int32)
    adj = adj32.reshape(2, NW, NCHUNK, CH)
    zeros = jnp.zeros((RPS, D), jnp.float32)
    zerosd = jnp.zeros((NPAD,), jnp.float32)

    parts, deg_parts = _sc_segment_sum(adj, table, zeros, zerosd)

    return pl.pallas_call(
        _tc_epilogue,
        out_shape=(
            jax.ShapeDtypeStruct((N_USER, D), jnp.float32),
            jax.ShapeDtypeStruct((N_ITEM, D), jnp.float32),
        ),
        in_specs=[
            pl.BlockSpec(memory_space=pltpu.VMEM),
            pl.BlockSpec(memory_space=pltpu.VMEM),
            pl.BlockSpec(memory_space=pltpu.VMEM),
            pl.BlockSpec(memory_space=pltpu.VMEM),
            pl.BlockSpec(memory_space=pltpu.VMEM),
            pl.BlockSpec(memory_space=pltpu.SMEM),
        ],
        out_specs=(
            pl.BlockSpec(memory_space=pltpu.VMEM),
            pl.BlockSpec(memory_space=pltpu.VMEM),
        ),
    )(xu_t_plus, xi_t_plus, parts, deg_parts, W, t_diff.reshape(1, 1))


# speedup vs baseline: 1.5921x; 1.0026x over previous
import functools

import jax
import jax.numpy as jnp
from jax import lax
from jax.experimental import pallas as pl
from jax.experimental.pallas import tpu as pltpu
from jax.experimental.pallas import tpu_sc as plsc

N_USER = 6000
N_ITEM = 4000
N = N_USER + N_ITEM
D = 128
E = 320000
NC = 2
NS = 16
NW = NC * NS
EPW = E // NW
CH = 80
NCHUNK = EPW // CH
NPAD = 10240
RPS = NPAD // NS

_mesh = plsc.VectorSubcoreMesh(core_axis_name="c", subcore_axis_name="s")


@functools.partial(
    pl.kernel,
    mesh=_mesh,
    compiler_params=pltpu.CompilerParams(needs_layout_passes=False),
    out_type=(
        jax.ShapeDtypeStruct((NC, NPAD, D), jnp.float32),
        jax.ShapeDtypeStruct((NW, NPAD), jnp.float32),
    ),
    scratch_types=[
        pltpu.VMEM((4, 2, CH), jnp.int32),
        pltpu.VMEM((2, CH, D), jnp.float32),
        pltpu.VMEM((NPAD,), jnp.float32),
        pltpu.VMEM_SHARED((NPAD, D), jnp.float32),
        pltpu.SemaphoreType.DMA,
        pltpu.SemaphoreType.DMA,
        pltpu.SemaphoreType.DMA,
        pltpu.SemaphoreType.DMA,
        pltpu.SemaphoreType.DMA,
        pltpu.SemaphoreType.DMA,
    ],
)
def _sc_segment_sum(adj_hbm, table_hbm, zeros_hbm, zerosd_hbm,
                    out_hbm, outdeg_hbm,
                    idx_v, rows_v, deg_v, acc_sh,
                    gsem0, gsem1, isem0, isem1, isem2, isem3):
    c = lax.axis_index("c")
    s = lax.axis_index("s")
    wid = c * NS + s

    ones16 = jnp.full((16,), 1.0, jnp.float32)
    gsems = (gsem0, gsem1)
    isems = (isem0, isem1, isem2, isem3)

    def fire_idx(j, m):
        pltpu.async_copy(adj_hbm.at[0, wid, j], idx_v.at[m, 0], isems[m])
        pltpu.async_copy(adj_hbm.at[1, wid, j], idx_v.at[m, 1], isems[m])

    def fire_gather(j, m, r):
        pltpu.make_async_copy(adj_hbm.at[0, wid, j], idx_v.at[m, 0], isems[m]).wait()
        pltpu.make_async_copy(adj_hbm.at[1, wid, j], idx_v.at[m, 1], isems[m]).wait()
        pltpu.async_copy(table_hbm.at[idx_v.at[m, 0]], rows_v.at[r], gsems[r])
        for k in range(CH // 16):
            idx16 = idx_v[m, 1, pl.ds(k * 16, 16)]
            plsc.addupdate_scatter(deg_v, [idx16], ones16)

    def finish(j, m, r):
        pltpu.make_async_copy(
            table_hbm.at[idx_v.at[m, 0]], rows_v.at[r], gsems[r]).wait()
        pltpu.sync_copy(rows_v.at[r], acc_sh.at[idx_v.at[m, 1]], add=True)

        @pl.when(j + 4 < NCHUNK)
        def _():
            fire_idx(j + 4, m)

    for m in range(4):
        fire_idx(m, m)
    pltpu.sync_copy(zeros_hbm, acc_sh.at[pl.ds(s * RPS, RPS)])
    pltpu.sync_copy(zerosd_hbm, deg_v)
    plsc.subcore_barrier()
    fire_gather(0, 0, 0)

    def body(k, carry):
        base = 4 * k
        fire_gather(base + 1, 1, 1)
        finish(base, 0, 0)
        fire_gather(base + 2, 2, 0)
        finish(base + 1, 1, 1)
        fire_gather(base + 3, 3, 1)
        finish(base + 2, 2, 0)
        fire_gather(base + 4, 0, 0)
        finish(base + 3, 3, 1)
        return carry

    lax.fori_loop(0, NCHUNK // 4, body, 0)
    finish(NCHUNK - 1, 0, 0)
    pltpu.sync_copy(deg_v, outdeg_hbm.at[wid])
    plsc.subcore_barrier()
    pltpu.sync_copy(acc_sh.at[pl.ds(s * RPS, RPS)],
                    out_hbm.at[c, pl.ds(s * RPS, RPS)])


def _tc_epilogue(xu_ref, xi_ref, parts_ref, degp_ref, w_ref, t_ref,
                 outu_ref, outi_ref):
    xu = xu_ref[...]
    xi = xi_ref[...]
    nrm2 = jnp.maximum(jnp.max(jnp.sum(xu * xu, axis=1)),
                       jnp.max(jnp.sum(xi * xi, axis=1)))
    inv_nrm = lax.rsqrt(nrm2)
    agg = parts_ref[0] + parts_ref[1]
    deg = jnp.sum(jnp.transpose(degp_ref[...]), axis=1, keepdims=True)
    scale = lax.rsqrt(deg + 1.0) * inv_nrm
    aggn = agg * scale
    w = w_ref[...]
    t = t_ref[0, 0]
    zu = jnp.tanh(jnp.dot(aggn[:N_USER], w,
                          preferred_element_type=jnp.float32))
    outu_ref[...] = xu * inv_nrm + t * zu
    zi = jnp.tanh(jnp.dot(aggn[N_USER:N], w,
                          preferred_element_type=jnp.float32))
    outi_ref[...] = xi * inv_nrm + t * zi


def kernel(adj_his, t_diff, xu_t_plus, xi_t_plus, xu_embed, xi_embed, W):
    table = jnp.concatenate([xu_embed, xi_embed], axis=0)
    adj32 = adj_his.astype(jnp.int32)
    adj = adj32.reshape(2, NW, NCHUNK, CH)
    zeros = jnp.zeros((RPS, D), jnp.float32)
    zerosd = jnp.zeros((NPAD,), jnp.float32)

    parts, deg_parts = _sc_segment_sum(adj, table, zeros, zerosd)

    return pl.pallas_call(
        _tc_epilogue,
        out_shape=(
            jax.ShapeDtypeStruct((N_USER, D), jnp.float32),
            jax.ShapeDtypeStruct((N_ITEM, D), jnp.float32),
        ),
        in_specs=[
            pl.BlockSpec(memory_space=pltpu.VMEM),
            pl.BlockSpec(memory_space=pltpu.VMEM),
            pl.BlockSpec(memory_space=pltpu.VMEM),
            pl.BlockSpec(memory_space=pltpu.VMEM),
            pl.BlockSpec(memory_space=pltpu.VMEM),
            pl.BlockSpec(memory_space=pltpu.SMEM),
        ],
        out_specs=(
            pl.BlockSpec(memory_space=pltpu.VMEM),
            pl.BlockSpec(memory_space=pltpu.VMEM),
        ),
    )(xu_t_plus, xi_t_plus, parts, deg_parts, W, t_diff.reshape(1, 1))


# speedup vs baseline: 1.5935x; 1.0008x over previous
import functools

import jax
import jax.numpy as jnp
from jax import lax
from jax.experimental import pallas as pl
from jax.experimental.pallas import tpu as pltpu
from jax.experimental.pallas import tpu_sc as plsc

N_USER = 6000
N_ITEM = 4000
N = N_USER + N_ITEM
D = 128
E = 320000
NC = 2
NS = 16
NW = NC * NS
EPW = E // NW
CH = 80
NCHUNK = EPW // CH
NPAD = 10240
RPS = NPAD // NS

_mesh = plsc.VectorSubcoreMesh(core_axis_name="c", subcore_axis_name="s")


@functools.partial(
    pl.kernel,
    mesh=_mesh,
    compiler_params=pltpu.CompilerParams(needs_layout_passes=False),
    out_type=(
        jax.ShapeDtypeStruct((NC, NPAD, D), jnp.float32),
        jax.ShapeDtypeStruct((NW, NPAD), jnp.float32),
    ),
    scratch_types=[
        pltpu.VMEM((4, 2, CH), jnp.int32),
        pltpu.VMEM((2, CH, D), jnp.float32),
        pltpu.VMEM((NPAD,), jnp.float32),
        pltpu.VMEM_SHARED((NPAD, D), jnp.float32),
        pltpu.SemaphoreType.DMA,
        pltpu.SemaphoreType.DMA,
        pltpu.SemaphoreType.DMA,
        pltpu.SemaphoreType.DMA,
        pltpu.SemaphoreType.DMA,
        pltpu.SemaphoreType.DMA,
    pltpu.SemaphoreType.DMA,
        pltpu.SemaphoreType.DMA,
    ],
)
def _sc_segment_sum(adj_hbm, table_hbm, zeros_hbm, zerosd_hbm,
                    out_hbm, outdeg_hbm,
                    idx_v, rows_v, deg_v, acc_sh,
                    gsem0, gsem1, ssem0, ssem1, isem0, isem1, isem2, isem3):
    c = lax.axis_index("c")
    s = lax.axis_index("s")
    wid = c * NS + s

    ones16 = jnp.full((16,), 1.0, jnp.float32)
    gsems = (gsem0, gsem1)
    ssems = (ssem0, ssem1)
    isems = (isem0, isem1, isem2, isem3)

    def fire_idx(j, m):
        pltpu.async_copy(adj_hbm.at[0, wid, j], idx_v.at[m, 0], isems[m])
        pltpu.async_copy(adj_hbm.at[1, wid, j], idx_v.at[m, 1], isems[m])

    def wait_scat(j, m, r):
        pltpu.make_async_copy(
            rows_v.at[r], acc_sh.at[idx_v.at[m, 1]], ssems[r]).wait()

    def fire_gather(j, m, r):
        pltpu.make_async_copy(adj_hbm.at[0, wid, j], idx_v.at[m, 0], isems[m]).wait()
        pltpu.make_async_copy(adj_hbm.at[1, wid, j], idx_v.at[m, 1], isems[m]).wait()

        @pl.when(j >= 2)
        def _():
            wait_scat(j - 2, (m + 2) % 4, r)

        pltpu.async_copy(table_hbm.at[idx_v.at[m, 0]], rows_v.at[r], gsems[r])

        @pl.when((j >= 2) & (j + 2 < NCHUNK))
        def _():
            fire_idx(j + 2, (m + 2) % 4)

        for k in range(CH // 16):
            idx16 = idx_v[m, 1, pl.ds(k * 16, 16)]
            plsc.addupdate_scatter(deg_v, [idx16], ones16)

    def finish(j, m, r):
        pltpu.make_async_copy(
            table_hbm.at[idx_v.at[m, 0]], rows_v.at[r], gsems[r]).wait()
        pltpu.async_copy(rows_v.at[r], acc_sh.at[idx_v.at[m, 1]], ssems[r],
                         add=True)

    for m in range(4):
        fire_idx(m, m)
    pltpu.sync_copy(zeros_hbm, acc_sh.at[pl.ds(s * RPS, RPS)])
    pltpu.sync_copy(zerosd_hbm, deg_v)
    plsc.subcore_barrier()
    fire_gather(0, 0, 0)

    def body(k, carry):
        base = 4 * k
        fire_gather(base + 1, 1, 1)
        finish(base, 0, 0)
        fire_gather(base + 2, 2, 0)
        finish(base + 1, 1, 1)
        fire_gather(base + 3, 3, 1)
        finish(base + 2, 2, 0)
        fire_gather(base + 4, 0, 0)
        finish(base + 3, 3, 1)
        return carry

    lax.fori_loop(0, NCHUNK // 4, body, 0)
    finish(NCHUNK - 1, 0, 0)
    wait_scat(NCHUNK - 2, 3, 1)
    wait_scat(NCHUNK - 1, 0, 0)
    pltpu.sync_copy(deg_v, outdeg_hbm.at[wid])
    plsc.subcore_barrier()
    pltpu.sync_copy(acc_sh.at[pl.ds(s * RPS, RPS)],
                    out_hbm.at[c, pl.ds(s * RPS, RPS)])


def _tc_epilogue(xu_ref, xi_ref, parts_ref, degp_ref, w_ref, t_ref,
                 outu_ref, outi_ref):
    xu = xu_ref[...]
    xi = xi_ref[...]
    nrm2 = jnp.maximum(jnp.max(jnp.sum(xu * xu, axis=1)),
                       jnp.max(jnp.sum(xi * xi, axis=1)))
    inv_nrm = lax.rsqrt(nrm2)
    agg = parts_ref[0] + parts_ref[1]
    deg = jnp.sum(jnp.transpose(degp_ref[...]), axis=1, keepdims=True)
    scale = lax.rsqrt(deg + 1.0) * inv_nrm
    aggn = agg * scale
    w = w_ref[...]
    t = t_ref[0, 0]
    zu = jnp.tanh(jnp.dot(aggn[:N_USER], w,
                          preferred_element_type=jnp.float32))
    outu_ref[...] = xu * inv_nrm + t * zu
    zi = jnp.tanh(jnp.dot(aggn[N_USER:N], w,
                          preferred_element_type=jnp.float32))
    outi_ref[...] = xi * inv_nrm + t * zi


def kernel(adj_his, t_diff, xu_t_plus, xi_t_plus, xu_embed, xi_embed, W):
    table = jnp.concatenate([xu_embed, xi_embed], axis=0)
    adj32 = adj_his.astype(jnp.int32)
    adj = adj32.reshape(2, NW, NCHUNK, CH)
    zeros = jnp.zeros((RPS, D), jnp.float32)
    zerosd = jnp.zeros((NPAD,), jnp.float32)

    parts, deg_parts = _sc_segment_sum(adj, table, zeros, zerosd)

    return pl.pallas_call(
        _tc_epilogue,
        out_shape=(
            jax.ShapeDtypeStruct((N_USER, D), jnp.float32),
            jax.ShapeDtypeStruct((N_ITEM, D), jnp.float32),
        ),
        in_specs=[
            pl.BlockSpec(memory_space=pltpu.VMEM),
            pl.BlockSpec(memory_space=pltpu.VMEM),
            pl.BlockSpec(memory_space=pltpu.VMEM),
            pl.BlockSpec(memory_space=pltpu.VMEM),
            pl.BlockSpec(memory_space=pltpu.VMEM),
            pl.BlockSpec(memory_space=pltpu.SMEM),
        ],
        out_specs=(
            pl.BlockSpec(memory_space=pltpu.VMEM),
            pl.BlockSpec(memory_space=pltpu.VMEM),
        ),
    )(xu_t_plus, xi_t_plus, parts, deg_parts, W, t_diff.reshape(1, 1))


# speedup vs baseline: 1.5956x; 1.0013x over previous
import functools

import jax
import jax.numpy as jnp
from jax import lax
from jax.experimental import pallas as pl
from jax.experimental.pallas import tpu as pltpu
from jax.experimental.pallas import tpu_sc as plsc

N_USER = 6000
N_ITEM = 4000
N = N_USER + N_ITEM
D = 128
E = 320000
NC = 2
NS = 16
NW = NC * NS
EPW = E // NW
CH = 80
NCHUNK = EPW // CH
NPAD = 10240
RPS = NPAD // NS

_mesh = plsc.VectorSubcoreMesh(core_axis_name="c", subcore_axis_name="s")


@functools.partial(
    pl.kernel,
    mesh=_mesh,
    compiler_params=pltpu.CompilerParams(needs_layout_passes=False),
    out_type=(
        jax.ShapeDtypeStruct((NC, NPAD, D), jnp.float32),
        jax.ShapeDtypeStruct((NW, NPAD), jnp.float32),
    ),
    scratch_types=[
        pltpu.VMEM((4, 2, CH), jnp.int32),
        pltpu.VMEM((2, CH, D), jnp.float32),
        pltpu.VMEM((NPAD,), jnp.float32),
        pltpu.VMEM_SHARED((NPAD, D), jnp.float32),
        pltpu.SemaphoreType.DMA,
        pltpu.SemaphoreType.DMA,
        pltpu.SemaphoreType.DMA,
        pltpu.SemaphoreType.DMA,
        pltpu.SemaphoreType.DMA,
        pltpu.SemaphoreType.DMA,
    ],
)
def _sc_segment_sum(adj_hbm, table_hbm, zeros_hbm, zerosd_hbm,
                    out_hbm, outdeg_hbm,
                    idx_v, rows_v, deg_v, acc_sh,
                    gsem0, gsem1, isem0, isem1, isem2, isem3):
    c = lax.axis_index("c")
    s = lax.axis_index("s")
    wid = c * NS + s

    ones16 = jnp.full((16,), 1.0, jnp.float32)
    gsems = (gsem0, gsem1)
    isems = (isem0, isem1, isem2, isem3)

    def fire_idx(j, m):
        pltpu.async_copy(adj_hbm.at[0, wid, j], idx_v.at[m, 0], isems[m])
        pltpu.async_copy(adj_hbm.at[1, wid, j], idx_v.at[m, 1], isems[m])

    def fire_gather(j, m, r):
        pltpu.make_async_copy(adj_hbm.at[0, wid, j], idx_v.at[m, 0], isems[m]).wait()
        pltpu.make_async_copy(adj_hbm.at[1, wid, j], idx_v.at[m, 1], isems[m]).wait()
        pltpu.async_copy(table_hbm.at[idx_v.at[m, 0]], rows_v.at[r], gsems[r])
        for k in range(CH // 16):
            idx16 = idx_v[m, 1, pl.ds(k * 16, 16)]
            plsc.addupdate_scatter(deg_v, [idx16], ones16)

    def finish(j, m, r):
        pltpu.make_async_copy(
            table_hbm.at[idx_v.at[m, 0]], rows_v.at[r], gsems[r]).wait()
        pltpu.sync_copy(rows_v.at[r], acc_sh.at[idx_v.at[m, 1]], add=True)

        @pl.when(j + 4 < NCHUNK)
        def _():
            fire_idx(j + 4, m)

    for m in range(4):
        fire_idx(m, m)
    pltpu.sync_copy(zeros_hbm, acc_sh.at[pl.ds(s * RPS, RPS)])
    pltpu.sync_copy(zerosd_hbm, deg_v)
    plsc.subcore_barrier()
    fire_gather(0, 0, 0)

    def body(k, carry):
        base = 4 * k
        fire_gather(base + 1, 1, 1)
        finish(base, 0, 0)
        fire_gather(base + 2, 2, 0)
        finish(base + 1, 1, 1)
        fire_gather(base + 3, 3, 1)
        finish(base + 2, 2, 0)
        fire_gather(base + 4, 0, 0)
        finish(base + 3, 3, 1)
        return carry

    lax.fori_loop(0, NCHUNK // 4, body, 0)
    finish(NCHUNK - 1, 0, 0)
    pltpu.sync_copy(deg_v, outdeg_hbm.at[wid])
    plsc.subcore_barrier()
    pltpu.sync_copy(acc_sh.at[pl.ds(s * RPS, RPS)],
                    out_hbm.at[c, pl.ds(s * RPS, RPS)])


def _tc_epilogue(xu_ref, xi_ref, parts_ref, degp_ref, w_ref, t_ref,
                 outu_ref, outi_ref):
    xu = xu_ref[...]
    xi = xi_ref[...]
    nrm2 = jnp.maximum(jnp.max(jnp.sum(xu * xu, axis=1)),
                       jnp.max(jnp.sum(xi * xi, axis=1)))
    inv_nrm = lax.rsqrt(nrm2)
    agg = parts_ref[0] + parts_ref[1]
    deg = jnp.sum(jnp.transpose(degp_ref[...]), axis=1, keepdims=True)
    scale = lax.rsqrt(deg + 1.0) * inv_nrm
    aggn = agg * scale
    w = w_ref[...]
    t = t_ref[0, 0]
    zu = jnp.tanh(jnp.dot(aggn[:N_USER], w,
                          preferred_element_type=jnp.float32))
    outu_ref[...] = xu * inv_nrm + t * zu
    zi = jnp.tanh(jnp.dot(aggn[N_USER:N], w,
                          preferred_element_type=jnp.float32))
    outi_ref[...] = xi * inv_nrm + t * zi


def kernel(adj_his, t_diff, xu_t_plus, xi_t_plus, xu_embed, xi_embed, W):
    table = jnp.concatenate([xu_embed, xi_embed], axis=0)
    adj32 = adj_his.astype(jnp.int32)
    adj = adj32.reshape(2, NW, NCHUNK, CH)
    zeros = jnp.zeros((RPS, D), jnp.float32)
    zerosd = jnp.zeros((NPAD,), jnp.float32)

    parts, deg_parts = _sc_segment_sum(adj, table, zeros, zerosd)

    return pl.pallas_call(
        _tc_epilogue,
        out_shape=(
            jax.ShapeDtypeStruct((N_USER, D), jnp.float32),
            jax.ShapeDtypeStruct((N_ITEM, D), jnp.float32),
        ),
        in_specs=[
            pl.BlockSpec(memory_space=pltpu.VMEM),
            pl.BlockSpec(memory_space=pltpu.VMEM),
            pl.BlockSpec(memory_space=pltpu.VMEM),
            pl.BlockSpec(memory_space=pltpu.VMEM),
            pl.BlockSpec(memory_space=pltpu.VMEM),
            pl.BlockSpec(memory_space=pltpu.SMEM),
        ],
        out_specs=(
            pl.BlockSpec(memory_space=pltpu.VMEM),
            pl.BlockSpec(memory_space=pltpu.VMEM),
        ),
    )(xu_t_plus, xi_t_plus, parts, deg_parts, W, t_diff.reshape(1, 1))
